# baseline (speedup 1.0000x reference)
import functools

import numpy as np
import jax
import jax.numpy as jnp
from jax.experimental import pallas as pl
from jax.experimental.pallas import tpu as pltpu


_RANGE_BASE = {"face": 0, "leftHand": 468, "pose": 489, "rightHand": 522,
               "root": 543}


def _gather_indices():
    hand_dfs = [0, 1, 2, 3, 4, 3, 2, 1, 0, 5, 6, 7, 8, 7, 6, 5, 0, 9, 10,
                11, 12, 11, 10, 9, 0, 13, 14, 15, 16, 15, 14, 13, 0, 17, 18,
                19, 20, 19, 18, 17, 0]
    order = ["root", "pose_11", "pose_13", "pose_15"]
    order += ["leftHand_%d" % i for i in hand_dfs]
    order += ["pose_15", "pose_13", "pose_11", "pose_12", "pose_14",
              "pose_16"]
    order += ["rightHand_%d" % i for i in hand_dfs]
    order += ["face_%d" % i for i in
              [61, 185, 40, 39, 37, 0, 267, 269, 270, 409, 291]]
    out = []
    for joint in order:
        kind = joint.split("_")[0]
        lid = 0 if kind == "root" else int(joint.split("_")[1])
        out.append(_RANGE_BASE[kind] + lid)
    return np.asarray(out, dtype=np.int32)


_IDX = _gather_indices()
_B, _T, _K, _C = 64, 200, 544, 3
_NP = _B * _C
_NOUT = len(_IDX)

_SEL = np.zeros((_K, _NOUT), dtype=np.float32)
_SEL[_IDX, np.arange(_NOUT)] = 1.0

_PLANES_PER_STEP = 32


def _body(x_ref, s_ref, o_ref):
    s = s_ref[...]
    for p in range(_PLANES_PER_STEP):
        o_ref[p] = jax.lax.dot_general(
            x_ref[p], s, (((1,), (0,)), ((), ())),
            preferred_element_type=jnp.float32)


@jax.jit
def _run(xp):
    grid = (_NP // _PLANES_PER_STEP,)
    return pl.pallas_call(
        _body,
        grid=grid,
        in_specs=[
            pl.BlockSpec((_PLANES_PER_STEP, _T, _K), lambda g: (g, 0, 0)),
            pl.BlockSpec((_K, _NOUT), lambda g: (0, 0)),
        ],
        out_specs=pl.BlockSpec((_PLANES_PER_STEP, _T, _NOUT),
                               lambda g: (g, 0, 0)),
        out_shape=jax.ShapeDtypeStruct((_NP, _T, _NOUT), jnp.float32),
        compiler_params=pltpu.CompilerParams(
            dimension_semantics=("parallel",)),
    )(xp, jnp.asarray(_SEL))


def kernel(keypoints):
    xp = jnp.transpose(keypoints, (0, 3, 1, 2)).reshape(_NP, _T, _K)
    out = _run(xp)
    return jnp.transpose(out.reshape(_B, _C, _T, _NOUT), (0, 2, 3, 1))


# speedup vs baseline: 1.0299x; 1.0299x over previous
import functools

import numpy as np
import jax
import jax.numpy as jnp
from jax.experimental import pallas as pl
from jax.experimental.pallas import tpu as pltpu


_RANGE_BASE = {"face": 0, "leftHand": 468, "pose": 489, "rightHand": 522,
               "root": 543}


def _gather_indices():
    hand_dfs = [0, 1, 2, 3, 4, 3, 2, 1, 0, 5, 6, 7, 8, 7, 6, 5, 0, 9, 10,
                11, 12, 11, 10, 9, 0, 13, 14, 15, 16, 15, 14, 13, 0, 17, 18,
                19, 20, 19, 18, 17, 0]
    order = ["root", "pose_11", "pose_13", "pose_15"]
    order += ["leftHand_%d" % i for i in hand_dfs]
    order += ["pose_15", "pose_13", "pose_11", "pose_12", "pose_14",
              "pose_16"]
    order += ["rightHand_%d" % i for i in hand_dfs]
    order += ["face_%d" % i for i in
              [61, 185, 40, 39, 37, 0, 267, 269, 270, 409, 291]]
    out = []
    for joint in order:
        kind = joint.split("_")[0]
        lid = 0 if kind == "root" else int(joint.split("_")[1])
        out.append(_RANGE_BASE[kind] + lid)
    return np.asarray(out, dtype=np.int32)


_IDX = _gather_indices()
_B, _T, _K, _C = 64, 200, 544, 3
_NP = _B * _C
_NOUT = len(_IDX)

_SEL = np.zeros((_K, _NOUT), dtype=np.float32)
_SEL[_IDX, np.arange(_NOUT)] = 1.0

_PLANES_PER_STEP = 16


def _body(x_ref, s_ref, o_ref):
    x = x_ref[...].reshape(_PLANES_PER_STEP * _T, _K)
    out = jax.lax.dot_general(
        x, s_ref[...], (((1,), (0,)), ((), ())),
        preferred_element_type=jnp.float32)
    o_ref[...] = out.reshape(_PLANES_PER_STEP, _T, _NOUT)


@jax.jit
def _run(xp):
    grid = (_NP // _PLANES_PER_STEP,)
    return pl.pallas_call(
        _body,
        grid=grid,
        in_specs=[
            pl.BlockSpec((_PLANES_PER_STEP, _T, _K), lambda g: (g, 0, 0)),
            pl.BlockSpec((_K, _NOUT), lambda g: (0, 0)),
        ],
        out_specs=pl.BlockSpec((_PLANES_PER_STEP, _T, _NOUT),
                               lambda g: (g, 0, 0)),
        out_shape=jax.ShapeDtypeStruct((_NP, _T, _NOUT), jnp.float32),
        compiler_params=pltpu.CompilerParams(
            dimension_semantics=("parallel",)),
    )(xp, jnp.asarray(_SEL))


def kernel(keypoints):
    xp = jnp.transpose(keypoints, (0, 3, 1, 2)).reshape(_NP, _T, _K)
    out = _run(xp)
    return jnp.transpose(out.reshape(_B, _C, _T, _NOUT), (0, 2, 3, 1))


# speedup vs baseline: 1.0334x; 1.0034x over previous
import functools

import numpy as np
import jax
import jax.numpy as jnp
from jax.experimental import pallas as pl
from jax.experimental.pallas import tpu as pltpu


_RANGE_BASE = {"face": 0, "leftHand": 468, "pose": 489, "rightHand": 522,
               "root": 543}


def _gather_indices():
    hand_dfs = [0, 1, 2, 3, 4, 3, 2, 1, 0, 5, 6, 7, 8, 7, 6, 5, 0, 9, 10,
                11, 12, 11, 10, 9, 0, 13, 14, 15, 16, 15, 14, 13, 0, 17, 18,
                19, 20, 19, 18, 17, 0]
    order = ["root", "pose_11", "pose_13", "pose_15"]
    order += ["leftHand_%d" % i for i in hand_dfs]
    order += ["pose_15", "pose_13", "pose_11", "pose_12", "pose_14",
              "pose_16"]
    order += ["rightHand_%d" % i for i in hand_dfs]
    order += ["face_%d" % i for i in
              [61, 185, 40, 39, 37, 0, 267, 269, 270, 409, 291]]
    out = []
    for joint in order:
        kind = joint.split("_")[0]
        lid = 0 if kind == "root" else int(joint.split("_")[1])
        out.append(_RANGE_BASE[kind] + lid)
    return np.asarray(out, dtype=np.int32)


_IDX = _gather_indices()
_B, _T, _K, _C = 64, 200, 544, 3
_NP = _B * _C
_NOUT = len(_IDX)

_SEL = np.zeros((_K, _NOUT), dtype=np.float32)
_SEL[_IDX, np.arange(_NOUT)] = 1.0

_PLANES_PER_STEP = 24


def _body(x_ref, s_ref, o_ref):
    x = x_ref[...].reshape(_PLANES_PER_STEP * _T, _K)
    out = jax.lax.dot_general(
        x, s_ref[...], (((1,), (0,)), ((), ())),
        preferred_element_type=jnp.float32)
    o_ref[...] = out.reshape(_PLANES_PER_STEP, _T, _NOUT)


@jax.jit
def _run(xp):
    grid = (_NP // _PLANES_PER_STEP,)
    return pl.pallas_call(
        _body,
        grid=grid,
        in_specs=[
            pl.BlockSpec((_PLANES_PER_STEP, _T, _K), lambda g: (g, 0, 0)),
            pl.BlockSpec((_K, _NOUT), lambda g: (0, 0)),
        ],
        out_specs=pl.BlockSpec((_PLANES_PER_STEP, _T, _NOUT),
                               lambda g: (g, 0, 0)),
        out_shape=jax.ShapeDtypeStruct((_NP, _T, _NOUT), jnp.float32),
        compiler_params=pltpu.CompilerParams(
            dimension_semantics=("parallel",)),
    )(xp, jnp.asarray(_SEL))


def kernel(keypoints):
    xp = jnp.transpose(keypoints, (0, 3, 1, 2)).reshape(_NP, _T, _K)
    out = _run(xp)
    return jnp.transpose(out.reshape(_B, _C, _T, _NOUT), (0, 2, 3, 1))
